# baseline (speedup 1.0000x reference)
import functools

import jax
import jax.numpy as jnp
from jax import lax
from jax.experimental import pallas as pl
from jax.experimental.pallas import tpu as pltpu
from jax.experimental.pallas import tpu_sc as plsc

_NUM_CORES = 2
_NUM_SUBCORES = 16
_NUM_WORKERS = _NUM_CORES * _NUM_SUBCORES
_CHUNK = 128


def _gather_rows(idx_grouped, table_pairs):
    n_chunks = idx_grouped.shape[1]
    b_per_w = n_chunks * _CHUNK
    batch = _NUM_WORKERS * b_per_w
    dim = table_pairs.shape[1]

    mesh = plsc.VectorSubcoreMesh(core_axis_name="c", subcore_axis_name="s")

    @functools.partial(
        pl.kernel,
        mesh=mesh,
        out_type=jax.ShapeDtypeStruct((batch, dim), jnp.float32),
        scratch_types=[
            pltpu.VMEM((n_chunks, _CHUNK), jnp.int32),
            pltpu.VMEM((b_per_w, dim), jnp.float32),
            pltpu.SemaphoreType.DMA,
        ],
    )
    def k(idx_hbm, table_hbm, out_hbm, idx_v, rows_v, sem):
        wid = lax.axis_index("s") * _NUM_CORES + lax.axis_index("c")
        base = wid * b_per_w
        pltpu.sync_copy(idx_hbm.at[wid], idx_v)
        copies = []
        for j in range(n_chunks):
            copies.append(
                pltpu.async_copy(
                    table_hbm.at[idx_v.at[j]],
                    rows_v.at[pl.ds(j * _CHUNK, _CHUNK)],
                    sem,
                )
            )
        for c in copies:
            c.wait()
        pltpu.sync_copy(rows_v, out_hbm.at[pl.ds(base, b_per_w)])

    return k(idx_grouped, table_pairs)


def kernel(preprocessed_states, table):
    idx = jnp.reshape(preprocessed_states.astype(jnp.int32), (-1,))
    table_pairs = jnp.reshape(table, (table.shape[0] // 2, 2 * table.shape[1]))
    rows = jnp.reshape(idx // 2, (_NUM_WORKERS, -1, _CHUNK))
    pairs = _gather_rows(rows, table_pairs)
    half = (idx % 2).astype(jnp.int32)
    dim = table.shape[1]
    out = jnp.where(
        (half == 0)[:, None], pairs[:, :dim], pairs[:, dim:]
    )
    return out


# speedup vs baseline: 1.0131x; 1.0131x over previous
import functools

import jax
import jax.numpy as jnp
from jax import lax
from jax.experimental import pallas as pl
from jax.experimental.pallas import tpu as pltpu
from jax.experimental.pallas import tpu_sc as plsc

_NUM_CORES = 2
_NUM_SUBCORES = 16
_NUM_WORKERS = _NUM_CORES * _NUM_SUBCORES
_CHUNK = 128


def _gather_rows(idx_grouped, table):
    n_chunks = idx_grouped.shape[1]
    b_per_w = n_chunks * _CHUNK
    batch = _NUM_WORKERS * b_per_w
    dim = table.shape[1]

    mesh = plsc.VectorSubcoreMesh(core_axis_name="c", subcore_axis_name="s")

    @functools.partial(
        pl.kernel,
        mesh=mesh,
        out_type=jax.ShapeDtypeStruct((batch, dim), jnp.float32),
        scratch_types=[
            pltpu.VMEM((n_chunks, _CHUNK), jnp.int32),
            pltpu.VMEM((b_per_w, dim), jnp.float32),
            pltpu.SemaphoreType.DMA,
        ],
        compiler_params=pltpu.CompilerParams(use_tc_tiling_on_sc=False),
    )
    def k(idx_hbm, table_hbm, out_hbm, idx_v, rows_v, sem):
        wid = lax.axis_index("s") * _NUM_CORES + lax.axis_index("c")
        base = wid * b_per_w
        pltpu.sync_copy(idx_hbm.at[wid], idx_v)
        copies = []
        for j in range(n_chunks):
            copies.append(
                pltpu.async_copy(
                    table_hbm.at[idx_v.at[j]],
                    rows_v.at[pl.ds(j * _CHUNK, _CHUNK)],
                    sem,
                )
            )
        for c in copies:
            c.wait()
        pltpu.sync_copy(rows_v, out_hbm.at[pl.ds(base, b_per_w)])

    return k(idx_grouped, table)


def kernel(preprocessed_states, table):
    idx = jnp.reshape(
        preprocessed_states.astype(jnp.int32),
        (_NUM_WORKERS, -1, _CHUNK),
    )
    return _gather_rows(idx, table)


# speedup vs baseline: 2.0521x; 2.0255x over previous
import functools

import jax
import jax.numpy as jnp
from jax import lax
from jax.experimental import pallas as pl
from jax.experimental.pallas import tpu as pltpu
from jax.experimental.pallas import tpu_sc as plsc

_NUM_CORES = 2
_NUM_SUBCORES = 16
_NUM_WORKERS = _NUM_CORES * _NUM_SUBCORES
_LANES = 16
_SLAB = 128
_NBUF = 4


def _gather_slabs(off_col, table_t):
    b_per_w = off_col.shape[2]
    batch = _NUM_WORKERS * b_per_w
    dim = table_t.shape[0]

    mesh = plsc.VectorSubcoreMesh(core_axis_name="c", subcore_axis_name="s")

    @functools.partial(
        pl.kernel,
        mesh=mesh,
        out_type=jax.ShapeDtypeStruct((batch, dim), jnp.float32),
        scratch_types=[
            pltpu.VMEM((b_per_w,), jnp.int32),
            pltpu.VMEM((b_per_w,), jnp.int32),
            pltpu.VMEM((_NBUF, dim, _SLAB), jnp.float32),
            pltpu.VMEM((b_per_w, dim), jnp.float32),
            pltpu.SemaphoreType.DMA,
        ],
        compiler_params=pltpu.CompilerParams(
            use_tc_tiling_on_sc=True, needs_layout_passes=False
        ),
    )
    def k(oc_hbm, table_hbm, out_hbm, off_v, col_v, bufs, rows_v, sem):
        wid = lax.axis_index("s") * _NUM_CORES + lax.axis_index("c")
        base = wid * b_per_w
        pltpu.sync_copy(oc_hbm.at[wid, 0], off_v)
        pltpu.sync_copy(oc_hbm.at[wid, 1], col_v)
        lanes = jax.lax.broadcasted_iota(jnp.int32, (_LANES,), 0)

        def body(g, _):
            voff = off_v[pl.ds(g * _LANES, _LANES)]
            vcol = col_v[pl.ds(g * _LANES, _LANES)]
            for h in range(_LANES // _NBUF):
                handles = []
                for r in range(_NBUF):
                    t = h * _NBUF + r
                    off = pl.multiple_of(voff[t], _SLAB)
                    handles.append(
                        pltpu.async_copy(
                            table_hbm.at[:, pl.ds(off, _SLAB)],
                            bufs.at[r],
                            sem,
                        )
                    )
                for cp in handles:
                    cp.wait()
                for r in range(_NBUF):
                    t = h * _NBUF + r
                    colv = jnp.zeros((_LANES,), jnp.int32) + vcol[t]
                    bv = jnp.zeros((_LANES,), jnp.int32) + (g * _LANES + t)
                    for c0 in range(0, dim, _LANES):
                        cv = lanes + c0
                        vals = plsc.load_gather(bufs.at[r], [cv, colv])
                        plsc.store_scatter(rows_v, [bv, cv], vals)
            return 0

        lax.fori_loop(0, b_per_w // _LANES, body, 0)
        pltpu.sync_copy(rows_v, out_hbm.at[pl.ds(base, b_per_w)])

    return k(off_col, table_t)


def kernel(preprocessed_states, table):
    idx = jnp.reshape(preprocessed_states.astype(jnp.int32), (-1,))
    off = (idx // _SLAB) * _SLAB
    col = idx - off
    off_col = jnp.stack(
        [
            jnp.reshape(off, (_NUM_WORKERS, -1)),
            jnp.reshape(col, (_NUM_WORKERS, -1)),
        ],
        axis=1,
    )
    table_t = jnp.transpose(table)
    return _gather_slabs(off_col, table_t)


# speedup vs baseline: 2.8177x; 1.3731x over previous
import functools

import jax
import jax.numpy as jnp
from jax import lax
from jax.experimental import pallas as pl
from jax.experimental.pallas import tpu as pltpu
from jax.experimental.pallas import tpu_sc as plsc

_NUM_CORES = 2
_NUM_SUBCORES = 16
_NUM_WORKERS = _NUM_CORES * _NUM_SUBCORES
_LANES = 16
_SLAB = 128
_NBUF = 4


def _gather_slabs(off_col, table_t):
    b_per_w = off_col.shape[2]
    batch = _NUM_WORKERS * b_per_w
    dim = table_t.shape[0]

    mesh = plsc.VectorSubcoreMesh(core_axis_name="c", subcore_axis_name="s")

    @functools.partial(
        pl.kernel,
        mesh=mesh,
        out_type=jax.ShapeDtypeStruct((batch, dim), jnp.float32),
        scratch_types=[
            pltpu.VMEM((b_per_w,), jnp.int32),
            pltpu.VMEM((b_per_w,), jnp.int32),
            pltpu.VMEM((2 * _NBUF, dim, _SLAB), jnp.float32),
            pltpu.VMEM((b_per_w // 2, dim), jnp.float32),
            pltpu.SemaphoreType.DMA,
        ],
        compiler_params=pltpu.CompilerParams(
            use_tc_tiling_on_sc=True, needs_layout_passes=False
        ),
    )
    def k(oc_hbm, table_hbm, out_hbm, off_v, col_v, bufs, rows_v, sem):
        wid = lax.axis_index("s") * _NUM_CORES + lax.axis_index("c")
        base = wid * b_per_w
        pltpu.sync_copy(oc_hbm.at[wid, 0], off_v)
        pltpu.sync_copy(oc_hbm.at[wid, 1], col_v)
        lanes = jax.lax.broadcasted_iota(jnp.int32, (_LANES,), 0)

        n_batches = _LANES // _NBUF

        def fire(voff, h, p):
            handles = []
            for r in range(_NBUF):
                off = pl.multiple_of(voff[h * _NBUF + r], _SLAB)
                handles.append(
                    pltpu.async_copy(
                        table_hbm.at[:, pl.ds(off, _SLAB)],
                        bufs.at[p * _NBUF + r],
                        sem,
                    )
                )
            return handles

        def select(vcol, g_local, h, p):
            for r in range(_NBUF):
                t = h * _NBUF + r
                colv = jnp.zeros((_LANES,), jnp.int32) + vcol[t]
                bv = jnp.zeros((_LANES,), jnp.int32) + (g_local * _LANES + t)
                for c0 in range(0, dim, _LANES):
                    cv = lanes + c0
                    vals = plsc.load_gather(bufs.at[p * _NBUF + r], [cv, colv])
                    plsc.store_scatter(rows_v, [bv, cv], vals)

        half_groups = b_per_w // (2 * _LANES)
        for half in range(2):

            def body(g, _, half=half):
                voff = off_v[pl.ds(g * _LANES, _LANES)]
                vcol = col_v[pl.ds(g * _LANES, _LANES)]
                pending = fire(voff, 0, 0)
                for h in range(n_batches):
                    nxt = None
                    if h + 1 < n_batches:
                        nxt = fire(voff, h + 1, (h + 1) % 2)
                    for cp in pending:
                        cp.wait()
                    select(vcol, g - half * half_groups, h, h % 2)
                    pending = nxt
                return 0

            lax.fori_loop(half * half_groups, (half + 1) * half_groups, body, 0)
            pltpu.sync_copy(
                rows_v,
                out_hbm.at[pl.ds(base + half * (b_per_w // 2), b_per_w // 2)],
            )

    return k(off_col, table_t)


def kernel(preprocessed_states, table):
    idx = jnp.reshape(preprocessed_states.astype(jnp.int32), (-1,))
    off = (idx // _SLAB) * _SLAB
    col = idx - off
    off_col = jnp.stack(
        [
            jnp.reshape(off, (_NUM_WORKERS, -1)),
            jnp.reshape(col, (_NUM_WORKERS, -1)),
        ],
        axis=1,
    )
    table_t = jnp.transpose(table)
    return _gather_slabs(off_col, table_t)
